# baseline (speedup 1.0000x reference)
import functools

import jax
import jax.numpy as jnp
import numpy as np
from jax import lax
from jax.experimental import pallas as pl
from jax.experimental.pallas import tpu as pltpu
from jax.experimental.pallas import tpu_sc as plsc

N_NODES = 10000
E_TOTAL = 320000
D = 128
D_E = 16

NC = 2
NS = 16
NW = NC * NS
E_PH = (32000, 64000, 96000, 128000)
B = 40
RSTRIDE = 624
RLEN = 640

BM = 2000
BE2 = 16000


def _node_body(x0_ref, x1_ref, we0_ref, be0_ref, wm_ref, we1_ref, be1_ref,
               ws_ref, g0_ref, z1_ref):
    h0 = jnp.dot(x0_ref[...], we0_ref[...],
                 preferred_element_type=jnp.float32) + be0_ref[...]
    g0_ref[...] = jnp.dot(h0, wm_ref[...], preferred_element_type=jnp.float32)
    h1 = jnp.dot(x1_ref[...], we1_ref[...],
                 preferred_element_type=jnp.float32) + be1_ref[...]
    z1_ref[...] = jnp.dot(h1, ws_ref[...], preferred_element_type=jnp.float32)


def _node_prep(x0, x1, W_emb0, b_emb0, W_msg, W_emb1, b_emb1, W_self):
    n = x0.shape[0]
    grid = n // BM
    row_spec = pl.BlockSpec((BM, D), lambda m: (m, 0))
    w_spec = pl.BlockSpec((D, D), lambda m: (0, 0))
    b_spec = pl.BlockSpec((1, D), lambda m: (0, 0))
    return pl.pallas_call(
        _node_body,
        grid=(grid,),
        in_specs=[row_spec, row_spec, w_spec, b_spec, w_spec, w_spec, b_spec,
                  w_spec],
        out_specs=[row_spec, row_spec],
        out_shape=[jax.ShapeDtypeStruct((n, D), jnp.float32),
                   jax.ShapeDtypeStruct((n, D), jnp.float32)],
    )(x0, x1, W_emb0, b_emb0.reshape(1, D), W_msg, W_emb1,
      b_emb1.reshape(1, D), W_self)


def _ea_body(attr_ref, we_ref, out_ref):
    out_ref[...] = jnp.dot(attr_ref[...], we_ref[...],
                           preferred_element_type=jnp.float32)


def _edge_attr_mm(edge_attr, W_edge):
    n = edge_attr.shape[0]
    grid = n // BE2
    return pl.pallas_call(
        _ea_body,
        grid=(grid,),
        in_specs=[pl.BlockSpec((BE2, D_E), lambda m: (m, 0)),
                  pl.BlockSpec((D_E, D), lambda m: (0, 0))],
        out_specs=pl.BlockSpec((BE2, D), lambda m: (m, 0)),
        out_shape=jax.ShapeDtypeStruct((n, D), jnp.float32),
    )(edge_attr, W_edge)


def _make_sc_body(steps):
  def _sc_body(g0, ea, idxh, zeros, out,
               sd, rows, eab, acc,
               isem, gsem0, gsem1, gsem2, esem0, esem1, esem2,
               ssem0, ssem1, ssem2):
    gsem = (gsem0, gsem1, gsem2)
    esem = (esem0, esem1, esem2)
    ssem = (ssem0, ssem1, ssem2)
    cid = lax.axis_index("c")
    sid = lax.axis_index("s")
    wid = cid * NS + sid
    base = wid * (steps * B)

    pltpu.sync_copy(zeros.at[pl.ds(sid * RSTRIDE, RLEN)],
                    acc.at[pl.ds(sid * RSTRIDE, RLEN)])
    plsc.subcore_barrier()

    def issue_idx(j):
        pltpu.async_copy(idxh.at[wid, j], sd.at[lax.rem(j, 4)], isem)

    def wait_idx():
        pltpu.make_async_copy(idxh.at[wid, 0], sd.at[0], isem).wait()

    def issue_ge(j, k):
        pltpu.async_copy(g0.at[sd.at[lax.rem(j, 4), 0]], rows.at[k], gsem[k])
        pltpu.async_copy(ea.at[pl.ds(base + j * B, B)], eab.at[k], esem[k])

    def wait_ge(k):
        pltpu.make_async_copy(g0.at[sd.at[0, 0]], rows.at[k], gsem[k]).wait()
        pltpu.make_async_copy(ea.at[pl.ds(0, B)], eab.at[k], esem[k]).wait()

    def issue_scatter(j, k):
        pltpu.async_copy(rows.at[k], acc.at[sd.at[lax.rem(j, 4), 1]],
                         ssem[k], add=True)

    def wait_scatter(k):
        pltpu.make_async_copy(rows.at[k], acc.at[sd.at[0, 1]], ssem[k]).wait()

    def compute(k):
        @plsc.parallel_loop(0, B, unroll=4)
        def rowfn(r):
            for c8 in range(D // 16):
                sl = pl.ds(c8 * 16, 16)
                rows[k, r, sl] = jnp.maximum(rows[k, r, sl] + eab[k, r, sl],
                                             0.0)

    def do_step(j, k, issue_next_ge, issue_next_idx, wait_sc):
        wait_ge(k)
        if wait_sc:
            wait_scatter((k + 1) % 3)
        if issue_next_ge:
            wait_idx()
            issue_ge(j + 1, (k + 1) % 3)
        if issue_next_idx:
            issue_idx(j + 2)
        compute(k)
        issue_scatter(j, k)

    issue_idx(0)
    wait_idx()
    issue_ge(0, 0)
    issue_idx(1)
    do_step(0, 0, True, True, False)
    do_step(1, 1, True, True, False)

    R = (steps - 4) // 3

    def round_body(g, c):
        j = 3 * g + 2
        do_step(j, 2, True, True, True)
        do_step(j + 1, 0, True, True, True)
        do_step(j + 2, 1, True, True, True)
        return c
    lax.fori_loop(0, R, round_body, 0)

    for j in range(3 * R + 2, steps):
        do_step(j, j % 3, j + 1 < steps, j + 2 < steps, True)
    wait_scatter((steps - 2) % 3)
    wait_scatter((steps - 1) % 3)

    plsc.subcore_barrier()
    pltpu.sync_copy(acc.at[pl.ds(sid * RSTRIDE, RLEN)],
                    out.at[cid, pl.ds(sid * RSTRIDE, RLEN)])

  return _sc_body


def _sc_segment(g0, ea, idxh):
    steps = idxh.shape[1]
    zeros = jnp.zeros((N_NODES, D), jnp.float32)
    mesh = plsc.VectorSubcoreMesh(core_axis_name="c", subcore_axis_name="s",
                                  num_cores=NC, num_subcores=NS)
    fn = pl.kernel(
        _make_sc_body(steps),
        out_type=jax.ShapeDtypeStruct((NC, N_NODES, D), jnp.float32),
        mesh=mesh,
        scratch_types=[
            pltpu.VMEM((4, 2, B), jnp.int32),
            pltpu.VMEM((3, B, D), jnp.float32),
            pltpu.VMEM((3, B, D), jnp.float32),
            pltpu.VMEM_SHARED((N_NODES, D), jnp.float32),
        ] + [pltpu.SemaphoreType.DMA] * 10,
    )
    return fn(g0, ea, idxh, zeros)


def _fin_body(*refs):
    aggs = refs[:len(E_PH)]
    z1_ref, wo_ref, bo_ref, out_ref = refs[len(E_PH):]
    a = sum(r[0] + r[1] for r in aggs)
    y = jnp.dot(a, wo_ref[...], preferred_element_type=jnp.float32)
    out_ref[...] = jnp.maximum(y + z1_ref[...] + bo_ref[...], 0.0)


def _final(aggs, z1, W_out, b_out):
    grid = N_NODES // BM
    agg_spec = pl.BlockSpec((NC, BM, D), lambda m: (0, m, 0))
    return pl.pallas_call(
        _fin_body,
        grid=(grid,),
        in_specs=[agg_spec] * len(E_PH) + [
            pl.BlockSpec((BM, D), lambda m: (m, 0)),
            pl.BlockSpec((D, D), lambda m: (0, 0)),
            pl.BlockSpec((1, D), lambda m: (0, 0))],
        out_specs=pl.BlockSpec((BM, D), lambda m: (m, 0)),
        out_shape=jax.ShapeDtypeStruct((N_NODES, D), jnp.float32),
    )(*aggs, z1, W_out, b_out.reshape(1, D))


def kernel(x0, x1, edge_index, edge_attr, W_emb0, b_emb0, W_emb1, b_emb1,
           W_msg, W_edge, W_self, W_out, b_out):
    src = edge_index[0].astype(jnp.int32)
    dst = edge_index[1].astype(jnp.int32)
    idxh, ea_parts = [], []
    e0 = 0
    for n_ph in E_PH:
        steps = n_ph // (NW * B)
        idxh.append(jnp.stack([src[e0:e0 + n_ph].reshape(NW, steps, B),
                               dst[e0:e0 + n_ph].reshape(NW, steps, B)],
                              axis=2))
        ea_parts.append(_edge_attr_mm(edge_attr[e0:e0 + n_ph], W_edge))
        e0 += n_ph
    g0, z1 = _node_prep(x0, x1, W_emb0, b_emb0, W_msg, W_emb1, b_emb1, W_self)
    aggs = []
    g0p = g0
    for p in range(len(E_PH)):
        if p > 0:
            g0p, ea_parts[p], idxh[p], aggs[p - 1] = lax.optimization_barrier(
                (g0p, ea_parts[p], idxh[p], aggs[p - 1]))
        aggs.append(_sc_segment(g0p, ea_parts[p], idxh[p]))
    return _final(aggs, z1, W_out, b_out)


# speedup vs baseline: 1.0429x; 1.0429x over previous
import functools

import jax
import jax.numpy as jnp
import numpy as np
from jax import lax
from jax.experimental import pallas as pl
from jax.experimental.pallas import tpu as pltpu
from jax.experimental.pallas import tpu_sc as plsc

N_NODES = 10000
E_TOTAL = 320000
D = 128
D_E = 16

NC = 2
NS = 16
NW = NC * NS
E_PH = (64000, 96000, 160000)
B = 40
RSTRIDE = 624
RLEN = 640

BM = 2000
BE2 = 16000


def _node_body(x0_ref, x1_ref, we0_ref, be0_ref, wm_ref, we1_ref, be1_ref,
               ws_ref, g0_ref, z1_ref):
    h0 = jnp.dot(x0_ref[...], we0_ref[...],
                 preferred_element_type=jnp.float32) + be0_ref[...]
    g0_ref[...] = jnp.dot(h0, wm_ref[...], preferred_element_type=jnp.float32)
    h1 = jnp.dot(x1_ref[...], we1_ref[...],
                 preferred_element_type=jnp.float32) + be1_ref[...]
    z1_ref[...] = jnp.dot(h1, ws_ref[...], preferred_element_type=jnp.float32)


def _node_prep(x0, x1, W_emb0, b_emb0, W_msg, W_emb1, b_emb1, W_self):
    n = x0.shape[0]
    grid = n // BM
    row_spec = pl.BlockSpec((BM, D), lambda m: (m, 0))
    w_spec = pl.BlockSpec((D, D), lambda m: (0, 0))
    b_spec = pl.BlockSpec((1, D), lambda m: (0, 0))
    return pl.pallas_call(
        _node_body,
        grid=(grid,),
        in_specs=[row_spec, row_spec, w_spec, b_spec, w_spec, w_spec, b_spec,
                  w_spec],
        out_specs=[row_spec, row_spec],
        out_shape=[jax.ShapeDtypeStruct((n, D), jnp.float32),
                   jax.ShapeDtypeStruct((n, D), jnp.float32)],
    )(x0, x1, W_emb0, b_emb0.reshape(1, D), W_msg, W_emb1,
      b_emb1.reshape(1, D), W_self)


def _ea_body(attr_ref, we_ref, out_ref):
    out_ref[...] = jnp.dot(attr_ref[...], we_ref[...],
                           preferred_element_type=jnp.float32)


def _edge_attr_mm(edge_attr, W_edge):
    n = edge_attr.shape[0]
    grid = n // BE2
    return pl.pallas_call(
        _ea_body,
        grid=(grid,),
        in_specs=[pl.BlockSpec((BE2, D_E), lambda m: (m, 0)),
                  pl.BlockSpec((D_E, D), lambda m: (0, 0))],
        out_specs=pl.BlockSpec((BE2, D), lambda m: (m, 0)),
        out_shape=jax.ShapeDtypeStruct((n, D), jnp.float32),
    )(edge_attr, W_edge)


def _make_sc_body(steps):
  def _sc_body(g0, ea, idxh, zeros, out,
               sd, rows, eab, acc,
               isem, gsem0, gsem1, gsem2, esem0, esem1, esem2,
               ssem0, ssem1, ssem2):
    gsem = (gsem0, gsem1, gsem2)
    esem = (esem0, esem1, esem2)
    ssem = (ssem0, ssem1, ssem2)
    cid = lax.axis_index("c")
    sid = lax.axis_index("s")
    wid = cid * NS + sid
    base = wid * (steps * B)

    pltpu.sync_copy(zeros.at[pl.ds(sid * RSTRIDE, RLEN)],
                    acc.at[pl.ds(sid * RSTRIDE, RLEN)])
    plsc.subcore_barrier()

    def issue_idx(j):
        pltpu.async_copy(idxh.at[wid, j], sd.at[lax.rem(j, 4)], isem)

    def wait_idx():
        pltpu.make_async_copy(idxh.at[wid, 0], sd.at[0], isem).wait()

    def issue_ge(j, k):
        pltpu.async_copy(g0.at[sd.at[lax.rem(j, 4), 0]], rows.at[k], gsem[k])
        pltpu.async_copy(ea.at[pl.ds(base + j * B, B)], eab.at[k], esem[k])

    def wait_ge(k):
        pltpu.make_async_copy(g0.at[sd.at[0, 0]], rows.at[k], gsem[k]).wait()
        pltpu.make_async_copy(ea.at[pl.ds(0, B)], eab.at[k], esem[k]).wait()

    def issue_scatter(j, k):
        pltpu.async_copy(rows.at[k], acc.at[sd.at[lax.rem(j, 4), 1]],
                         ssem[k], add=True)

    def wait_scatter(k):
        pltpu.make_async_copy(rows.at[k], acc.at[sd.at[0, 1]], ssem[k]).wait()

    def compute(k):
        @plsc.parallel_loop(0, B, unroll=4)
        def rowfn(r):
            for c8 in range(D // 16):
                sl = pl.ds(c8 * 16, 16)
                rows[k, r, sl] = jnp.maximum(rows[k, r, sl] + eab[k, r, sl],
                                             0.0)

    def do_step(j, k, issue_next_ge, issue_next_idx, wait_sc):
        wait_ge(k)
        if wait_sc:
            wait_scatter((k + 1) % 3)
        if issue_next_ge:
            wait_idx()
            issue_ge(j + 1, (k + 1) % 3)
        if issue_next_idx:
            issue_idx(j + 2)
        compute(k)
        issue_scatter(j, k)

    issue_idx(0)
    wait_idx()
    issue_ge(0, 0)
    issue_idx(1)
    do_step(0, 0, True, True, False)
    do_step(1, 1, True, True, False)

    R = (steps - 4) // 3

    def round_body(g, c):
        j = 3 * g + 2
        do_step(j, 2, True, True, True)
        do_step(j + 1, 0, True, True, True)
        do_step(j + 2, 1, True, True, True)
        return c
    lax.fori_loop(0, R, round_body, 0)

    for j in range(3 * R + 2, steps):
        do_step(j, j % 3, j + 1 < steps, j + 2 < steps, True)
    wait_scatter((steps - 2) % 3)
    wait_scatter((steps - 1) % 3)

    plsc.subcore_barrier()
    pltpu.sync_copy(acc.at[pl.ds(sid * RSTRIDE, RLEN)],
                    out.at[cid, pl.ds(sid * RSTRIDE, RLEN)])

  return _sc_body


def _sc_segment(g0, ea, idxh):
    steps = idxh.shape[1]
    zeros = jnp.zeros((N_NODES, D), jnp.float32)
    mesh = plsc.VectorSubcoreMesh(core_axis_name="c", subcore_axis_name="s",
                                  num_cores=NC, num_subcores=NS)
    fn = pl.kernel(
        _make_sc_body(steps),
        out_type=jax.ShapeDtypeStruct((NC, N_NODES, D), jnp.float32),
        mesh=mesh,
        scratch_types=[
            pltpu.VMEM((4, 2, B), jnp.int32),
            pltpu.VMEM((3, B, D), jnp.float32),
            pltpu.VMEM((3, B, D), jnp.float32),
            pltpu.VMEM_SHARED((N_NODES, D), jnp.float32),
        ] + [pltpu.SemaphoreType.DMA] * 10,
    )
    return fn(g0, ea, idxh, zeros)


def _fin_body(*refs):
    aggs = refs[:len(E_PH)]
    z1_ref, wo_ref, bo_ref, out_ref = refs[len(E_PH):]
    a = sum(r[0] + r[1] for r in aggs)
    y = jnp.dot(a, wo_ref[...], preferred_element_type=jnp.float32)
    out_ref[...] = jnp.maximum(y + z1_ref[...] + bo_ref[...], 0.0)


def _final(aggs, z1, W_out, b_out):
    grid = N_NODES // BM
    agg_spec = pl.BlockSpec((NC, BM, D), lambda m: (0, m, 0))
    return pl.pallas_call(
        _fin_body,
        grid=(grid,),
        in_specs=[agg_spec] * len(E_PH) + [
            pl.BlockSpec((BM, D), lambda m: (m, 0)),
            pl.BlockSpec((D, D), lambda m: (0, 0)),
            pl.BlockSpec((1, D), lambda m: (0, 0))],
        out_specs=pl.BlockSpec((BM, D), lambda m: (m, 0)),
        out_shape=jax.ShapeDtypeStruct((N_NODES, D), jnp.float32),
    )(*aggs, z1, W_out, b_out.reshape(1, D))


def kernel(x0, x1, edge_index, edge_attr, W_emb0, b_emb0, W_emb1, b_emb1,
           W_msg, W_edge, W_self, W_out, b_out):
    src = edge_index[0].astype(jnp.int32)
    dst = edge_index[1].astype(jnp.int32)
    idxh, ea_parts = [], []
    e0 = 0
    for n_ph in E_PH:
        steps = n_ph // (NW * B)
        idxh.append(jnp.stack([src[e0:e0 + n_ph].reshape(NW, steps, B),
                               dst[e0:e0 + n_ph].reshape(NW, steps, B)],
                              axis=2))
        ea_parts.append(_edge_attr_mm(edge_attr[e0:e0 + n_ph], W_edge))
        e0 += n_ph
    g0, z1 = _node_prep(x0, x1, W_emb0, b_emb0, W_msg, W_emb1, b_emb1, W_self)
    aggs = []
    g0p = g0
    for p in range(len(E_PH)):
        if p > 0:
            g0p, ea_parts[p], idxh[p], aggs[p - 1] = lax.optimization_barrier(
                (g0p, ea_parts[p], idxh[p], aggs[p - 1]))
        aggs.append(_sc_segment(g0p, ea_parts[p], idxh[p]))
    return _final(aggs, z1, W_out, b_out)
